# Initial kernel scaffold; baseline (speedup 1.0000x reference)
import jax
import jax.numpy as jnp
from jax.experimental import pallas as pl


def kernel(d2_0, d2_1, d2_2, w2, w3, w4, w9):
    raise NotImplementedError("write your pallas kernel here")


# speedup vs baseline: 1.6503x; 1.6503x over previous
import functools

import jax
import jax.numpy as jnp
from jax.experimental import pallas as pl
from jax.experimental.pallas import tpu as pltpu

_OFFSETS = tuple(
    [dx for dx in range(1, 5)]
    + [dy * 32 + dx for dy in range(1, 5) for dx in range(-4, 5)
       if dx * dx + dy * dy < 25]
)

_NPIX = 1024
_NSPAN = 892
_NK = len(_OFFSETS)


def _elu(v):
    return jnp.where(v > 0, v, jnp.exp(v) - 1.0)


def _f2_body(x0_ref, w2_ref, mt_ref, out_ref):
    f2 = jnp.dot(w2_ref[...], x0_ref[0], preferred_element_type=jnp.float32)
    f2 = _elu(f2)
    out_ref[0] = jnp.dot(f2, mt_ref[...], preferred_element_type=jnp.float32)


def _main_body(x1_ref, x2_ref, f2r_ref, w3_ref, w4_ref, w9a_ref, w9b_ref,
               w9c_ref, out_ref):
    f3 = _elu(jnp.dot(w3_ref[...], x1_ref[0],
                      preferred_element_type=jnp.float32))
    f4 = _elu(jnp.dot(w4_ref[...], x2_ref[0],
                      preferred_element_type=jnp.float32))
    acc = jnp.dot(w9a_ref[...], f2r_ref[0], preferred_element_type=jnp.float32)
    acc += jnp.dot(w9b_ref[...], f3, preferred_element_type=jnp.float32)
    acc += jnp.dot(w9c_ref[...], f4, preferred_element_type=jnp.float32)
    out_ref[0] = _elu(acc)


def _aff_body(x_ref, out_ref):
    x = x_ref[0]
    anchor = x[:, 0:_NSPAN]
    scale = jnp.full((1, x.shape[0]), 1.0 / x.shape[0], dtype=jnp.float32)
    for k, dk in enumerate(_OFFSETS):
        d = jnp.abs(x[:, dk:dk + _NSPAN] - anchor)
        e = jnp.dot(scale, d, preferred_element_type=jnp.float32)
        out_ref[0, k, 0:_NSPAN] = jnp.exp(-e)[0]


@functools.partial(jax.jit, static_argnums=())
def kernel(d2_0, d2_1, d2_2, w2, w3, w4, w9):
    B = d2_0.shape[0]
    f32 = jnp.float32
    X0 = d2_0.reshape(B, 512, 4096)
    X1 = d2_1.reshape(B, 1024, _NPIX)
    X2 = d2_2.reshape(B, 2048, _NPIX)

    R = jax.image.resize(jnp.eye(64, dtype=f32), (32, 64), method="bilinear")
    MT = jnp.kron(R, R).T

    f2r = pl.pallas_call(
        _f2_body,
        grid=(B,),
        in_specs=[
            pl.BlockSpec((1, 512, 4096), lambda b: (b, 0, 0)),
            pl.BlockSpec((64, 512), lambda b: (0, 0)),
            pl.BlockSpec((4096, 1024), lambda b: (0, 0)),
        ],
        out_specs=pl.BlockSpec((1, 64, _NPIX), lambda b: (b, 0, 0)),
        out_shape=jax.ShapeDtypeStruct((B, 64, _NPIX), f32),
    )(X0, w2, MT)

    x = pl.pallas_call(
        _main_body,
        grid=(B,),
        in_specs=[
            pl.BlockSpec((1, 1024, _NPIX), lambda b: (b, 0, 0)),
            pl.BlockSpec((1, 2048, _NPIX), lambda b: (b, 0, 0)),
            pl.BlockSpec((1, 64, _NPIX), lambda b: (b, 0, 0)),
            pl.BlockSpec((128, 1024), lambda b: (0, 0)),
            pl.BlockSpec((320, 2048), lambda b: (0, 0)),
            pl.BlockSpec((512, 64), lambda b: (0, 0)),
            pl.BlockSpec((512, 128), lambda b: (0, 0)),
            pl.BlockSpec((512, 320), lambda b: (0, 0)),
        ],
        out_specs=pl.BlockSpec((1, 512, _NPIX), lambda b: (b, 0, 0)),
        out_shape=jax.ShapeDtypeStruct((B, 512, _NPIX), f32),
    )(X1, X2, f2r, w3, w4, w9[:, 0:64], w9[:, 64:192], w9[:, 192:512])

    aff_full = pl.pallas_call(
        _aff_body,
        grid=(B,),
        in_specs=[pl.BlockSpec((1, 512, _NPIX), lambda b: (b, 0, 0))],
        out_specs=pl.BlockSpec((1, _NK, 896), lambda b: (b, 0, 0)),
        out_shape=jax.ShapeDtypeStruct((B, _NK, 896), f32),
    )(x)

    aff = aff_full.reshape(B, _NK, 28, 32)[:, :, :, 4:28]
    return aff.reshape(B, _NK, 672)


# speedup vs baseline: 1.9928x; 1.2075x over previous
import jax
import jax.numpy as jnp
from jax.experimental import pallas as pl
from jax.experimental.pallas import tpu as pltpu

_OFFSETS = tuple(
    [dx for dx in range(1, 5)]
    + [dy * 32 + dx for dy in range(1, 5) for dx in range(-4, 5)
       if dx * dx + dy * dy < 25]
)

_NPIX = 1024
_NSPAN = 892
_NK = len(_OFFSETS)


def _elu(v):
    return jnp.where(v > 0, v, jnp.exp(v) - 1.0)


def _f2_body(x0_ref, w2_ref, mt_ref, out_ref):
    x0 = x0_ref[0].astype(jnp.bfloat16)
    f2 = jnp.dot(w2_ref[...], x0, preferred_element_type=jnp.float32)
    f2 = _elu(f2).astype(jnp.bfloat16)
    out_ref[0] = jnp.dot(f2, mt_ref[...], preferred_element_type=jnp.float32)


def _main_body(x1_ref, x2_ref, f2r_ref, w3_ref, w4_ref, w9a_ref, w9b_ref,
               w9c_ref, out_ref):
    f3 = _elu(jnp.dot(w3_ref[...], x1_ref[0].astype(jnp.bfloat16),
                      preferred_element_type=jnp.float32)).astype(jnp.bfloat16)
    f4 = _elu(jnp.dot(w4_ref[...], x2_ref[0].astype(jnp.bfloat16),
                      preferred_element_type=jnp.float32)).astype(jnp.bfloat16)
    acc = jnp.dot(w9a_ref[...], f2r_ref[0].astype(jnp.bfloat16),
                  preferred_element_type=jnp.float32)
    acc += jnp.dot(w9b_ref[...], f3, preferred_element_type=jnp.float32)
    acc += jnp.dot(w9c_ref[...], f4, preferred_element_type=jnp.float32)
    out_ref[0] = _elu(acc)


def _aff_body(x_ref, out_ref):
    x = x_ref[0].astype(jnp.bfloat16)
    anchor = x[:, 0:_NSPAN]
    scale = jnp.full((1, x.shape[0]), 1.0 / x.shape[0], dtype=jnp.bfloat16)
    for k, dk in enumerate(_OFFSETS):
        d = jnp.abs(x[:, dk:dk + _NSPAN] - anchor)
        e = jnp.dot(scale, d, preferred_element_type=jnp.float32)
        out_ref[0, k, 0:_NSPAN] = jnp.exp(-e)[0]


def kernel(d2_0, d2_1, d2_2, w2, w3, w4, w9):
    B = d2_0.shape[0]
    f32 = jnp.float32
    bf16 = jnp.bfloat16
    X0 = d2_0.reshape(B, 512, 4096)
    X1 = d2_1.reshape(B, 1024, _NPIX)
    X2 = d2_2.reshape(B, 2048, _NPIX)

    R = jax.image.resize(jnp.eye(64, dtype=f32), (32, 64), method="bilinear")
    MT = jnp.kron(R, R).T.astype(bf16)

    f2r = pl.pallas_call(
        _f2_body,
        grid=(B,),
        in_specs=[
            pl.BlockSpec((1, 512, 4096), lambda b: (b, 0, 0)),
            pl.BlockSpec((64, 512), lambda b: (0, 0)),
            pl.BlockSpec((4096, 1024), lambda b: (0, 0)),
        ],
        out_specs=pl.BlockSpec((1, 64, _NPIX), lambda b: (b, 0, 0)),
        out_shape=jax.ShapeDtypeStruct((B, 64, _NPIX), f32),
    )(X0, w2.astype(bf16), MT)

    x = pl.pallas_call(
        _main_body,
        grid=(B,),
        in_specs=[
            pl.BlockSpec((1, 1024, _NPIX), lambda b: (b, 0, 0)),
            pl.BlockSpec((1, 2048, _NPIX), lambda b: (b, 0, 0)),
            pl.BlockSpec((1, 64, _NPIX), lambda b: (b, 0, 0)),
            pl.BlockSpec((128, 1024), lambda b: (0, 0)),
            pl.BlockSpec((320, 2048), lambda b: (0, 0)),
            pl.BlockSpec((512, 64), lambda b: (0, 0)),
            pl.BlockSpec((512, 128), lambda b: (0, 0)),
            pl.BlockSpec((512, 320), lambda b: (0, 0)),
        ],
        out_specs=pl.BlockSpec((1, 512, _NPIX), lambda b: (b, 0, 0)),
        out_shape=jax.ShapeDtypeStruct((B, 512, _NPIX), f32),
    )(X1, X2, f2r, w3.astype(bf16), w4.astype(bf16),
      w9[:, 0:64].astype(bf16), w9[:, 64:192].astype(bf16),
      w9[:, 192:512].astype(bf16))

    aff_full = pl.pallas_call(
        _aff_body,
        grid=(B,),
        in_specs=[pl.BlockSpec((1, 512, _NPIX), lambda b: (b, 0, 0))],
        out_specs=pl.BlockSpec((1, _NK, 896), lambda b: (b, 0, 0)),
        out_shape=jax.ShapeDtypeStruct((B, _NK, 896), f32),
    )(x)

    aff = aff_full.reshape(B, _NK, 28, 32)[:, :, :, 4:28]
    return aff.reshape(B, _NK, 672)
